# baseline (device time: 8757 ns/iter reference)
import jax
import jax.numpy as jnp
from jax import lax
from jax.experimental import pallas as pl
from jax.experimental.pallas import tpu as pltpu

C = 64


def kernel(x, dest):
    m, n = x.shape
    maxc = m // C

    def body(x_ref, dest_ref, out_ref, send_buf, recv_buf, send_sems, recv_sems):
        my_x = lax.axis_index("x")
        my_y = lax.axis_index("y")
        nbr = (my_x, 1 - my_y)

        recv_buf[...] = jnp.zeros((m, n), jnp.bfloat16)

        barrier_sem = pltpu.get_barrier_semaphore()
        pl.semaphore_signal(
            barrier_sem, inc=1, device_id=nbr, device_id_type=pl.DeviceIdType.MESH
        )

        x_bf = x_ref[...].astype(jnp.bfloat16)
        d = dest_ref[...]
        mkeep = d == my_y
        mkeep_bf = mkeep.astype(jnp.bfloat16)

        row = lax.broadcasted_iota(jnp.int32, (m, m), 0)
        col = lax.broadcasted_iota(jnp.int32, (m, m), 1)
        tri = (row < col).astype(jnp.bfloat16)

        before_keep = jnp.dot(
            mkeep_bf, tri, preferred_element_type=jnp.float32
        ).astype(jnp.int32)
        kept = jnp.sum(mkeep_bf.astype(jnp.float32)).astype(jnp.int32)
        K = m - kept

        col1 = lax.broadcasted_iota(jnp.int32, (1, m), 1)
        before_send = col1 - before_keep
        p_send = ((row == before_send) & jnp.logical_not(mkeep)).astype(jnp.bfloat16)
        send_buf[...] = jnp.dot(
            p_send, x_bf, preferred_element_type=jnp.float32
        ).astype(jnp.bfloat16)

        pl.semaphore_wait(barrier_sem, 1)

        rdmas = [
            pltpu.make_async_remote_copy(
                src_ref=send_buf.at[pl.ds(h * C, C)],
                dst_ref=recv_buf.at[pl.ds(h * C, C)],
                send_sem=send_sems.at[h],
                recv_sem=recv_sems.at[h],
                device_id=nbr,
                device_id_type=pl.DeviceIdType.MESH,
            )
            for h in range(maxc)
        ]
        for h in range(maxc):

            @pl.when(h * C < K)
            def _(h=h):
                rdmas[h].start()

        keep_off = jnp.where(my_y == 0, 0, K)
        recv_off = jnp.where(my_y == 0, kept, 0)
        p_keep = ((row == before_keep + keep_off) & mkeep).astype(jnp.bfloat16)
        acc = jnp.dot(p_keep, x_bf, preferred_element_type=jnp.float32)
        p_recv = ((row - col == recv_off) & (col < K)).astype(jnp.bfloat16)

        for h in range(maxc):

            @pl.when(h * C < K)
            def _(h=h):
                rdmas[h].wait_recv()

        acc = acc + jnp.dot(p_recv, recv_buf[...], preferred_element_type=jnp.float32)
        out_ref[...] = acc.astype(jnp.bfloat16)

        for h in range(maxc):

            @pl.when(h * C < K)
            def _(h=h):
                rdmas[h].wait_send()

    dest2 = dest.reshape(1, m)
    return pl.pallas_call(
        body,
        out_shape=jax.ShapeDtypeStruct((m, n), jnp.bfloat16),
        in_specs=[
            pl.BlockSpec(memory_space=pltpu.VMEM),
            pl.BlockSpec(memory_space=pltpu.VMEM),
        ],
        out_specs=pl.BlockSpec(memory_space=pltpu.VMEM),
        scratch_shapes=[
            pltpu.VMEM((m, n), jnp.bfloat16),
            pltpu.VMEM((m, n), jnp.bfloat16),
            pltpu.SemaphoreType.DMA((maxc,)),
            pltpu.SemaphoreType.DMA((maxc,)),
        ],
        compiler_params=pltpu.CompilerParams(collective_id=0),
    )(x, dest2)


# device time: 8742 ns/iter; 1.0017x vs baseline; 1.0017x over previous
import jax
import jax.numpy as jnp
from jax import lax
from jax.experimental import pallas as pl
from jax.experimental.pallas import tpu as pltpu

C = 64


def kernel(x, dest):
    m, n = x.shape
    maxc = m // C

    def body(x_ref, dest_ref, out_ref, send_buf, recv_buf, send_sems, recv_sems):
        my_x = lax.axis_index("x")
        my_y = lax.axis_index("y")
        nbr = (my_x, 1 - my_y)

        recv_buf[...] = jnp.zeros((m, n), jnp.bfloat16)

        barrier_sem = pltpu.get_barrier_semaphore()
        pl.semaphore_signal(
            barrier_sem, inc=1, device_id=nbr, device_id_type=pl.DeviceIdType.MESH
        )

        x_bf = x_ref[...].astype(jnp.bfloat16)
        d = dest_ref[...][None, :]
        mkeep = d == my_y
        mkeep_bf = mkeep.astype(jnp.bfloat16)

        row = lax.broadcasted_iota(jnp.int32, (m, m), 0)
        col = lax.broadcasted_iota(jnp.int32, (m, m), 1)
        tri = (row < col).astype(jnp.bfloat16)

        before_keep = jnp.dot(
            mkeep_bf, tri, preferred_element_type=jnp.float32
        ).astype(jnp.int32)
        kept = jnp.sum(mkeep_bf.astype(jnp.float32)).astype(jnp.int32)
        K = m - kept

        col1 = lax.broadcasted_iota(jnp.int32, (1, m), 1)
        before_send = col1 - before_keep
        p_send = ((row == before_send) & jnp.logical_not(mkeep)).astype(jnp.bfloat16)
        send_buf[...] = jnp.dot(
            p_send, x_bf, preferred_element_type=jnp.float32
        ).astype(jnp.bfloat16)

        pl.semaphore_wait(barrier_sem, 1)

        rdmas = [
            pltpu.make_async_remote_copy(
                src_ref=send_buf.at[pl.ds(h * C, C)],
                dst_ref=recv_buf.at[pl.ds(h * C, C)],
                send_sem=send_sems.at[h],
                recv_sem=recv_sems.at[h],
                device_id=nbr,
                device_id_type=pl.DeviceIdType.MESH,
            )
            for h in range(maxc)
        ]
        for h in range(maxc):

            @pl.when(h * C < K)
            def _(h=h):
                rdmas[h].start()

        keep_off = jnp.where(my_y == 0, 0, K)
        recv_off = jnp.where(my_y == 0, kept, 0)
        p_keep = ((row == before_keep + keep_off) & mkeep).astype(jnp.bfloat16)
        acc = jnp.dot(p_keep, x_bf, preferred_element_type=jnp.float32)
        p_recv = ((row - col == recv_off) & (col < K)).astype(jnp.bfloat16)

        for h in range(maxc):

            @pl.when(h * C < K)
            def _(h=h):
                rdmas[h].wait_recv()

        acc = acc + jnp.dot(p_recv, recv_buf[...], preferred_element_type=jnp.float32)
        out_ref[...] = acc.astype(jnp.bfloat16)

        for h in range(maxc):

            @pl.when(h * C < K)
            def _(h=h):
                rdmas[h].wait_send()

    return pl.pallas_call(
        body,
        out_shape=jax.ShapeDtypeStruct((m, n), jnp.bfloat16),
        in_specs=[
            pl.BlockSpec(memory_space=pltpu.VMEM),
            pl.BlockSpec(memory_space=pltpu.VMEM),
        ],
        out_specs=pl.BlockSpec(memory_space=pltpu.VMEM),
        scratch_shapes=[
            pltpu.VMEM((m, n), jnp.bfloat16),
            pltpu.VMEM((m, n), jnp.bfloat16),
            pltpu.SemaphoreType.DMA((maxc,)),
            pltpu.SemaphoreType.DMA((maxc,)),
        ],
        compiler_params=pltpu.CompilerParams(collective_id=0),
    )(x, dest)


# device time: 8523 ns/iter; 1.0275x vs baseline; 1.0257x over previous
import jax
import jax.numpy as jnp
from jax import lax
from jax.experimental import pallas as pl
from jax.experimental.pallas import tpu as pltpu

C = 64


def kernel(x, dest):
    m, n = x.shape
    maxc = m // C

    def body(x_ref, dest_ref, out_ref, send_buf, recv_buf, send_sems, recv_sems):
        my_x = lax.axis_index("x")
        my_y = lax.axis_index("y")
        nbr = (my_x, 1 - my_y)

        recv_buf[...] = jnp.zeros((m, n), jnp.bfloat16)

        barrier_sem = pltpu.get_barrier_semaphore()
        pl.semaphore_signal(
            barrier_sem, inc=1, device_id=nbr, device_id_type=pl.DeviceIdType.MESH
        )

        x_bf = x_ref[...].astype(jnp.bfloat16)
        d = dest_ref[...][None, :]
        mkeep = d == my_y
        mkeep_bf = mkeep.astype(jnp.bfloat16)

        row = lax.broadcasted_iota(jnp.int32, (m, m), 0)
        col = lax.broadcasted_iota(jnp.int32, (m, m), 1)
        tri = (row < col).astype(jnp.bfloat16)

        before_keep = jnp.dot(
            mkeep_bf, tri, preferred_element_type=jnp.float32
        ).astype(jnp.int32)
        kept = jnp.sum(mkeep_bf.astype(jnp.float32)).astype(jnp.int32)
        K = m - kept

        col1 = lax.broadcasted_iota(jnp.int32, (1, m), 1)
        before_send = col1 - before_keep
        not_keep = jnp.logical_not(mkeep)

        pl.semaphore_wait(barrier_sem, 1)

        rdmas = [
            pltpu.make_async_remote_copy(
                src_ref=send_buf.at[pl.ds(h * C, C)],
                dst_ref=recv_buf.at[pl.ds(h * C, C)],
                send_sem=send_sems.at[h],
                recv_sem=recv_sems.at[h],
                device_id=nbr,
                device_id_type=pl.DeviceIdType.MESH,
            )
            for h in range(maxc)
        ]

        for h in range(maxc):
            rowh = h * C + lax.broadcasted_iota(jnp.int32, (C, m), 0)
            p_send_h = ((rowh == before_send) & not_keep).astype(jnp.bfloat16)
            send_buf[pl.ds(h * C, C)] = jnp.dot(
                p_send_h, x_bf, preferred_element_type=jnp.float32
            ).astype(jnp.bfloat16)

            @pl.when(h * C < K)
            def _(h=h):
                rdmas[h].start()

        keep_off = jnp.where(my_y == 0, 0, K)
        recv_off = jnp.where(my_y == 0, kept, 0)
        p_keep = ((row == before_keep + keep_off) & mkeep).astype(jnp.bfloat16)
        acc = jnp.dot(p_keep, x_bf, preferred_element_type=jnp.float32)

        for h in range(maxc):

            @pl.when(h * C < K)
            def _(h=h):
                rdmas[h].wait_recv()

        shifted = pltpu.roll(recv_buf[...], recv_off, 0)
        out_ref[...] = (acc + shifted.astype(jnp.float32)).astype(jnp.bfloat16)

        for h in range(maxc):

            @pl.when(h * C < K)
            def _(h=h):
                rdmas[h].wait_send()

    return pl.pallas_call(
        body,
        out_shape=jax.ShapeDtypeStruct((m, n), jnp.bfloat16),
        in_specs=[
            pl.BlockSpec(memory_space=pltpu.VMEM),
            pl.BlockSpec(memory_space=pltpu.VMEM),
        ],
        out_specs=pl.BlockSpec(memory_space=pltpu.VMEM),
        scratch_shapes=[
            pltpu.VMEM((m, n), jnp.bfloat16),
            pltpu.VMEM((m, n), jnp.bfloat16),
            pltpu.SemaphoreType.DMA((maxc,)),
            pltpu.SemaphoreType.DMA((maxc,)),
        ],
        compiler_params=pltpu.CompilerParams(collective_id=0),
    )(x, dest)
